# baseline (speedup 1.0000x reference)
import functools
import math

import jax
import jax.numpy as jnp
from jax import lax
from jax.experimental import pallas as pl
from jax.experimental.pallas import tpu as pltpu
from jax.experimental.pallas import tpu_sc as plsc

D_MODEL = 64
SCALE = math.sqrt(D_MODEL)

NC = 2
NS = 16
NW = NC * NS
L = 16

CB = 128
NBUF = 4
LEAD = 2


def _make_emb_kernel(B, n_chunks):
    b_per_w = n_chunks * CB
    mesh = plsc.VectorSubcoreMesh(core_axis_name="c", subcore_axis_name="s")

    @functools.partial(
        pl.kernel,
        mesh=mesh,
        out_type=jax.ShapeDtypeStruct((B // 2, 128), jnp.float32),
        scratch_types=[
            pltpu.VMEM((n_chunks, CB), jnp.int32),
            pltpu.VMEM((n_chunks, CB), jnp.int32),
            pltpu.VMEM((NBUF, CB, 128), jnp.float32),
            pltpu.VMEM((NBUF, CB // 2, 128), jnp.float32),
        ]
        + [pltpu.SemaphoreType.DMA] * (2 * NBUF),
    )
    def emb(x_hbm, tpair_hbm, out_hbm, idx_v, hidx_v, pairs_v, st_v, *sems):
        gsems = sems[:NBUF]
        ssems = sems[NBUF:]
        wid = lax.axis_index("s") * NC + lax.axis_index("c")
        base2 = wid * (b_per_w // 2)
        pltpu.sync_copy(x_hbm.at[wid], idx_v)

        @plsc.parallel_loop(0, n_chunks, 1, unroll=2)
        def _(g):
            for c in range(CB // L):
                sl = pl.ds(c * L, L)
                hidx_v[g, sl] = jax.lax.shift_right_logical(idx_v[g, sl], 1)

        def start_gather(g, slot):
            pltpu.make_async_copy(
                tpair_hbm.at[hidx_v.at[g]], pairs_v.at[slot], gsems[slot]
            ).start()

        def wait_gather(g, slot):
            pltpu.make_async_copy(
                tpair_hbm.at[hidx_v.at[g]], pairs_v.at[slot], gsems[slot]
            ).wait()

        def start_scatter(g, slot):
            pltpu.make_async_copy(
                st_v.at[slot],
                out_hbm.at[pl.ds(base2 + g * (CB // 2), CB // 2)],
                ssems[slot],
            ).start()

        def wait_scatter(slot):
            pltpu.make_async_copy(
                st_v.at[slot], out_hbm.at[pl.ds(base2, CB // 2)], ssems[slot]
            ).wait()

        def select_scale(g, slot):
            pv = pairs_v.at[slot]
            sv = st_v.at[slot]

            @plsc.parallel_loop(0, CB, L)
            def _(r16):
                xv = idx_v[g, pl.ds(r16, L)]
                r8 = lax.div(r16, 2)
                for k in range(L):
                    off = (xv[k] & 1) * D_MODEL
                    src = r16 + k
                    dst = r8 + k // 2
                    doff = (k % 2) * D_MODEL
                    for c in range(D_MODEL // L):
                        sv[dst, pl.ds(doff + c * L, L)] = (
                            pv[src, pl.ds(off + c * L, L)] * SCALE
                        )

        def process(g, slot):
            wait_gather(g, slot)
            select_scale(g, slot)
            start_scatter(g, slot)

        start_gather(0, 0)
        start_gather(1, 1)
        process(0, 0)
        start_gather(2, 2)
        process(1, 1)
        start_gather(3, 3)

        def outer(go, carry):
            g0 = 2 + go * NBUF
            for bi in range(NBUF):
                g = g0 + bi
                slot = (2 + bi) % NBUF
                nslot = (slot + LEAD) % NBUF
                process(g, slot)
                wait_scatter(nslot)
                start_gather(g + LEAD, nslot)
            return carry

        lax.fori_loop(0, (n_chunks - NBUF) // NBUF, outer, 0)

        process(n_chunks - 2, (n_chunks - 2) % NBUF)
        process(n_chunks - 1, (n_chunks - 1) % NBUF)
        for s in range(NBUF):
            wait_scatter(s)

    return emb


def kernel(x, table):
    B0, B1 = x.shape
    B = B0 * B1
    assert B % (NW * CB) == 0
    n_chunks = B // (NW * CB)
    assert n_chunks % NBUF == 0 and n_chunks >= 2 * NBUF
    V = table.shape[0]
    xw = x.reshape(NW, n_chunks, CB).astype(jnp.int32)
    tpair = table.reshape(V // 2, 2 * D_MODEL)
    emb = _make_emb_kernel(B, n_chunks)
    out = emb(xw, tpair)
    return out.reshape(B0, B1, D_MODEL)


# speedup vs baseline: 1.0618x; 1.0618x over previous
import functools
import math

import jax
import jax.numpy as jnp
from jax import lax
from jax.experimental import pallas as pl
from jax.experimental.pallas import tpu as pltpu
from jax.experimental.pallas import tpu_sc as plsc

D_MODEL = 64
SCALE = math.sqrt(D_MODEL)

NC = 2
NS = 16
NW = NC * NS
L = 16

CB = 128
NBUF = 4
LEAD = 2


def _make_emb_kernel(B0, B1):
    bw = B0 // NW
    n_chunks = (bw * B1) // CB
    per_t = bw // CB
    mesh = plsc.VectorSubcoreMesh(core_axis_name="c", subcore_axis_name="s")

    @functools.partial(
        pl.kernel,
        mesh=mesh,
        compiler_params=pltpu.CompilerParams(needs_layout_passes=False),
        out_type=jax.ShapeDtypeStruct((B1, D_MODEL, B0), jnp.float32),
        scratch_types=[
            pltpu.VMEM((B1, bw), jnp.int32),
            pltpu.VMEM((n_chunks, CB), jnp.int32),
            pltpu.VMEM((NBUF, CB, 128), jnp.float32),
            pltpu.VMEM((NBUF, D_MODEL, CB), jnp.float32),
        ]
        + [pltpu.SemaphoreType.DMA] * (2 * NBUF),
    )
    def emb(xt_hbm, tpair_hbm, out_hbm, idx_v, hidx_v, pairs_v, st_v, *sems):
        gsems = sems[:NBUF]
        ssems = sems[NBUF:]
        wid = lax.axis_index("s") * NC + lax.axis_index("c")
        b0 = wid * bw
        pltpu.sync_copy(xt_hbm.at[:, pl.ds(b0, bw)], idx_v)

        @plsc.parallel_loop(0, (B1 * bw) // (16 * L), 1)
        def _(q):
            t = lax.div(q, 2)
            j_half = lax.rem(q, 2) * L
            for k in range(L):
                j0 = (j_half + k) * L
                row = 2 * q + k // 8
                lane = (k % 8) * L
                hidx_v[row, pl.ds(lane, L)] = lax.shift_right_logical(
                    idx_v[t, pl.ds(j0, L)], 1
                )

        def start_gather(g, slot):
            pltpu.make_async_copy(
                tpair_hbm.at[hidx_v.at[g]], pairs_v.at[slot], gsems[slot]
            ).start()

        def wait_gather(g, slot):
            pltpu.make_async_copy(
                tpair_hbm.at[hidx_v.at[g]], pairs_v.at[slot], gsems[slot]
            ).wait()

        def start_scatter(g, slot):
            t = lax.div(g, per_t)
            bj = b0 + lax.rem(g, per_t) * CB
            pltpu.make_async_copy(
                st_v.at[slot], out_hbm.at[t, :, pl.ds(bj, CB)], ssems[slot]
            ).start()

        def wait_scatter(slot):
            pltpu.make_async_copy(
                st_v.at[slot], out_hbm.at[0, :, pl.ds(b0, CB)], ssems[slot]
            ).wait()

        iota16 = lax.broadcasted_iota(jnp.int32, (L,), 0)

        def select_scale(g, slot):
            pv = pairs_v.at[slot]
            sv = st_v.at[slot]
            t = lax.div(g, per_t)
            jbase = lax.rem(g, per_t) * CB

            @plsc.parallel_loop(0, CB, L)
            def _(l0):
                offv = (idx_v[t, pl.ds(jbase + l0, L)] & 1) * D_MODEL
                rows = lax.broadcast(l0, (L,)) + iota16
                for c in range(D_MODEL):
                    vals = plsc.load_gather(pv, [rows, offv + c])
                    sv[c, pl.ds(l0, L)] = vals * SCALE

        def process(g, slot):
            wait_gather(g, slot)
            select_scale(g, slot)
            start_scatter(g, slot)

        start_gather(0, 0)
        start_gather(1, 1)
        process(0, 0)
        start_gather(2, 2)
        process(1, 1)
        start_gather(3, 3)

        def outer(go, carry):
            g0 = 2 + go * NBUF
            for bi in range(NBUF):
                g = g0 + bi
                slot = (2 + bi) % NBUF
                nslot = (slot + LEAD) % NBUF
                process(g, slot)
                wait_scatter(nslot)
                start_gather(g + LEAD, nslot)
            return carry

        lax.fori_loop(0, (n_chunks - NBUF) // NBUF, outer, 0)

        process(n_chunks - 2, (n_chunks - 2) % NBUF)
        process(n_chunks - 1, (n_chunks - 1) % NBUF)
        for s in range(NBUF):
            wait_scatter(s)

    return emb


def kernel(x, table):
    B0, B1 = x.shape
    assert B0 % (NW * CB) == 0 and (B0 // NW) % CB == 0
    V = table.shape[0]
    xt = x.T.astype(jnp.int32)
    tpair = table.reshape(V // 2, 2 * D_MODEL)
    emb = _make_emb_kernel(B0, B1)
    out = emb(xt, tpair)
    return jnp.transpose(out, (2, 0, 1))


# speedup vs baseline: 1.1697x; 1.1017x over previous
import functools
import math

import jax
import jax.numpy as jnp
from jax import lax
from jax.experimental import pallas as pl
from jax.experimental.pallas import tpu as pltpu
from jax.experimental.pallas import tpu_sc as plsc

D_MODEL = 64
SCALE = math.sqrt(D_MODEL)

NC = 2
NS = 16
NW = NC * NS
L = 16

NBUF = 4
LEAD = 2


def _make_emb_kernel(B0, B1):
    bw = B0 // NW
    bc = 4
    cb = bc * B1
    n_chunks = bw // bc
    row = B1 * D_MODEL
    mesh = plsc.VectorSubcoreMesh(core_axis_name="c", subcore_axis_name="s")

    @functools.partial(
        pl.kernel,
        mesh=mesh,
        compiler_params=pltpu.CompilerParams(use_tc_tiling_on_sc=False),
        out_type=jax.ShapeDtypeStruct((B0, row), jnp.float32),
        scratch_types=[
            pltpu.VMEM((n_chunks, cb), jnp.int32),
            pltpu.VMEM((NBUF, cb, D_MODEL), jnp.float32),
            pltpu.VMEM((NBUF, bc, row), jnp.float32),
        ]
        + [pltpu.SemaphoreType.DMA] * (2 * NBUF),
    )
    def emb(x_hbm, table_hbm, out_hbm, idx_v, rows_v, st_v, *sems):
        gsems = sems[:NBUF]
        ssems = sems[NBUF:]
        wid = lax.axis_index("s") * NC + lax.axis_index("c")
        b0 = wid * bw
        pltpu.sync_copy(x_hbm.at[wid], idx_v)

        def start_gather(g, slot):
            pltpu.make_async_copy(
                table_hbm.at[idx_v.at[g]], rows_v.at[slot], gsems[slot]
            ).start()

        def wait_gather(g, slot):
            pltpu.make_async_copy(
                table_hbm.at[idx_v.at[g]], rows_v.at[slot], gsems[slot]
            ).wait()

        def start_scatter(g, slot):
            pltpu.make_async_copy(
                st_v.at[slot], out_hbm.at[pl.ds(b0 + g * bc, bc)], ssems[slot]
            ).start()

        def wait_scatter(slot):
            pltpu.make_async_copy(
                st_v.at[slot], out_hbm.at[pl.ds(b0, bc)], ssems[slot]
            ).wait()

        def scale_rearrange(slot):
            rv = rows_v.at[slot]
            sv = st_v.at[slot]

            @plsc.parallel_loop(0, B1, 1, unroll=2)
            def _(t):
                for bi in range(bc):
                    for c in range(D_MODEL // L):
                        sv[bi, pl.ds(t * D_MODEL + c * L, L)] = (
                            rv[bi * B1 + t, pl.ds(c * L, L)] * SCALE
                        )

        def process(g, slot):
            wait_gather(g, slot)
            scale_rearrange(slot)
            start_scatter(g, slot)

        start_gather(0, 0)
        start_gather(1, 1)
        process(0, 0)
        start_gather(2, 2)
        process(1, 1)
        start_gather(3, 3)

        def outer(go, carry):
            g0 = 2 + go * NBUF
            for bi in range(NBUF):
                g = g0 + bi
                slot = (2 + bi) % NBUF
                nslot = (slot + LEAD) % NBUF
                process(g, slot)
                wait_scatter(nslot)
                start_gather(g + LEAD, nslot)
            return carry

        lax.fori_loop(0, (n_chunks - NBUF) // NBUF, outer, 0)

        process(n_chunks - 2, (n_chunks - 2) % NBUF)
        process(n_chunks - 1, (n_chunks - 1) % NBUF)
        for s in range(NBUF):
            wait_scatter(s)

    return emb


def kernel(x, table):
    B0, B1 = x.shape
    assert B0 % NW == 0 and (B0 // NW) % 4 == 0
    xw = x.reshape(NW, (B0 // NW) // 4, 4 * B1).astype(jnp.int32)
    emb = _make_emb_kernel(B0, B1)
    out = emb(xw, table)
    return out.reshape(B0, B1, D_MODEL)
